# Initial kernel scaffold; baseline (speedup 1.0000x reference)
import jax
import jax.numpy as jnp
from jax.experimental import pallas as pl


def kernel(images, features, conv_w, conv_b, cls_w, cls_b, reg_w, reg_b):
    raise NotImplementedError("write your pallas kernel here")


# speedup vs baseline: 20.3901x; 20.3901x over previous
import functools

import numpy as np
import jax
import jax.numpy as jnp
from jax import lax
from jax.experimental import pallas as pl
from jax.experimental.pallas import tpu as pltpu
from jax.experimental.pallas import tpu_sc as plsc

_SIZES = (64.0, 256.0, 576.0)
_RATIOS = (0.5, 1.0, 2.0)
_MIN_LEN = 10.0
_NMS_IOU = 0.7
_K = 2000
_KP = 2048
_A = 9
_P = 1024
_N = _A * _P
_NP = 16384
_BLK = 256


def _roll(x, shift, axis):
    return pltpu.roll(x, shift, axis)


def _anchor_consts(gh, gw, img_h, img_w):
    size = np.asarray(_SIZES, np.float32)
    ratio = np.asarray(_RATIOS, np.float32)
    h_ratio = np.sqrt(ratio)
    w_ratio = (np.float32(1.0) / h_ratio).astype(np.float32)
    hs = (h_ratio[:, None] * size[None, :]).reshape(-1)
    ws = (w_ratio[:, None] * size[None, :]).reshape(-1)
    base = np.round(np.stack([-ws, -hs, ws, hs], axis=1) / np.float32(2.0))
    base = base.astype(np.float32)
    stride_h = np.float32(img_h // gh)
    stride_w = np.float32(img_w // gw)
    sx = np.arange(gw, dtype=np.float32) * stride_w
    sy = np.arange(gh, dtype=np.float32) * stride_h
    yy, xx = np.meshgrid(sy, sx, indexing="ij")
    xx = xx.reshape(-1)
    yy = yy.reshape(-1)
    shifts = np.stack([xx, yy, xx, yy], axis=1).astype(np.float32)
    anchors = (shifts[:, None, :] + base[None, :, :]).reshape(-1, 4)
    anchors = anchors.astype(np.float32)
    anc = anchors.reshape(_P, _A, 4).transpose(1, 0, 2)
    x1, y1, x2, y2 = anc[..., 0], anc[..., 1], anc[..., 2], anc[..., 3]
    aw = (x2 - x1) + np.float32(1.0)
    ah = (y2 - y1) + np.float32(1.0)
    acx = x1 + np.float32(0.5) * aw
    acy = y1 + np.float32(0.5) * ah
    return aw, ah, acx, acy


def _head_body(x_ref, w9_ref, cb_ref, whd_ref, hb_ref,
               aw_ref, ah_ref, acx_ref, acy_ref,
               key_ref, prop_ref, *, img_h, img_w):
    y = lax.dot_general(w9_ref[:].astype(jnp.bfloat16),
                        x_ref[0].astype(jnp.bfloat16),
                        dimension_numbers=(((1,), (0,)), ((), ())),
                        preferred_element_type=jnp.float32)
    y = jnp.maximum(y + cb_ref[:], 0.0)
    heads = lax.dot_general(whd_ref[:].astype(jnp.bfloat16),
                            y.astype(jnp.bfloat16),
                            dimension_numbers=(((1,), (0,)), ((), ())),
                            preferred_element_type=jnp.float32) + hb_ref[:]
    for a in range(_A):
        x0 = heads[2 * a:2 * a + 1, :]
        x1 = heads[2 * a + 1:2 * a + 2, :]
        r = 18 + 4 * a
        dx = heads[r:r + 1, :]
        dy = heads[r + 1:r + 2, :]
        dw = heads[r + 2:r + 3, :]
        dh = heads[r + 3:r + 4, :]
        aw = aw_ref[pl.ds(a, 1), :]
        ah = ah_ref[pl.ds(a, 1), :]
        acx = acx_ref[pl.ds(a, 1), :]
        acy = acy_ref[pl.ds(a, 1), :]
        pcx = dx * aw + acx
        pcy = dy * ah + acy
        pw = jnp.exp(dw) * aw
        ph = jnp.exp(dh) * ah
        px1 = jnp.maximum(pcx - 0.5 * pw, 0.0)
        py1 = jnp.maximum(pcy - 0.5 * ph, 0.0)
        px2 = jnp.maximum(pcx + 0.5 * pw, 0.0)
        py2 = jnp.maximum(pcy + 0.5 * ph, 0.0)
        cx1 = jnp.clip(px1, 0.0, float(img_w))
        cx2 = jnp.clip(px2, 0.0, float(img_w))
        cy1 = jnp.clip(py1, 0.0, float(img_h))
        cy2 = jnp.clip(py2, 0.0, float(img_h))
        keep = ((cx2 - cx1) >= _MIN_LEN) & ((cy2 - cy1) >= _MIN_LEN)
        m = jnp.maximum(x0, x1)
        e0 = jnp.exp(x0 - m)
        e1 = jnp.exp(x1 - m)
        score = e1 / (e0 + e1)
        key_ref[0, pl.ds(a, 1), :] = jnp.where(keep, score, -1e9)
        prop_ref[0, 0, pl.ds(a, 1), :] = px1
        prop_ref[0, 1, pl.ds(a, 1), :] = py1
        prop_ref[0, 2, pl.ds(a, 1), :] = px2
        prop_ref[0, 3, pl.ds(a, 1), :] = py2


def _sort_body(k_ref, skey_ref, spos_ref, kbuf, cbuf):
    q = lax.broadcasted_iota(jnp.int32, (8, _NP), 1)
    orig = (q % _P) * _A + (q // _P)
    kbuf[:] = k_ref[:]
    cbuf[:] = orig * _NP + q

    def level(lvl, _):
        kk = lax.shift_left(1, lvl)

        def step_cond(j):
            return j >= 1

        def step(j):
            K = kbuf[:]
            C = cbuf[:]
            up_k = _roll(K, j, 1)
            dn_k = _roll(K, _NP - j, 1)
            up_c = _roll(C, j, 1)
            dn_c = _roll(C, _NP - j, 1)
            bit0 = (q & j) == 0
            pk = jnp.where(bit0, dn_k, up_k)
            pc = jnp.where(bit0, dn_c, up_c)
            self_first = (K > pk) | ((K == pk) & (C < pc))
            dirup = (q & kk) == 0
            take_self = self_first == (bit0 == dirup)
            kbuf[:] = jnp.where(take_self, K, pk)
            cbuf[:] = jnp.where(take_self, C, pc)
            return j // 2

        lax.while_loop(step_cond, step, kk // 2)
        return 0

    lax.fori_loop(1, 15, level, 0)
    skey_ref[:] = kbuf[:, :_KP]
    spos_ref[:] = cbuf[:, :_KP] & (_NP - 1)


_GCHUNK = 128


def _sc_gather_body(tab_hbm, posg_hbm, out_hbm, idx_v, rows_v, sem):
    c = lax.axis_index("c")
    s = lax.axis_index("s")
    wid = s * 2 + c
    per_w = (8 * _KP) // 32
    for g in range(per_w // _GCHUNK):
        base = wid * per_w + g * _GCHUNK
        pltpu.sync_copy(posg_hbm.at[pl.ds(base, _GCHUNK)], idx_v)
        pltpu.async_copy(tab_hbm.at[idx_v], rows_v, sem).wait()
        pltpu.sync_copy(rows_v, out_hbm.at[pl.ds(base, _GCHUNK)])


def _gather_topk(propsf, spos):
    B = spos.shape[0]
    tab = jnp.transpose(propsf, (0, 2, 1))
    tab = jnp.pad(tab, ((0, 0), (0, 0), (0, 12))).reshape(B * _N, 16)
    posg = (spos + (jnp.arange(B, dtype=jnp.int32) * _N)[:, None]).reshape(-1)
    mesh = plsc.VectorSubcoreMesh(core_axis_name="c", subcore_axis_name="s")
    run = pl.kernel(
        _sc_gather_body,
        out_type=jax.ShapeDtypeStruct((B * _KP, 16), jnp.float32),
        mesh=mesh,
        compiler_params=pltpu.CompilerParams(use_tc_tiling_on_sc=False),
        scratch_types=[
            pltpu.VMEM((_GCHUNK,), jnp.int32),
            pltpu.VMEM((_GCHUNK, 16), jnp.float32),
            pltpu.SemaphoreType.DMA,
        ],
    )
    rows = run(tab, posg)
    return jnp.transpose(rows.reshape(B, _KP, 16)[:, :, :4], (0, 2, 1))


def _nms_body(box_ref, key_ref, out_ref, act_ref):
    x1 = box_ref[0, 0:1, :]
    y1 = box_ref[0, 1:2, :]
    x2 = box_ref[0, 2:3, :]
    y2 = box_ref[0, 3:4, :]
    area = (x2 - x1) * (y2 - y1)
    jlane = lax.broadcasted_iota(jnp.int32, (1, _KP), 1)
    act_ref[:] = jnp.where(jlane < _K, 1.0, 0.0)

    def colchunk(v, j0):
        return v[:, j0:j0 + _BLK].reshape(_BLK, 1)

    def iou_chunk(j0, i0):
        xj1 = colchunk(x1, j0)
        yj1 = colchunk(y1, j0)
        xj2 = colchunk(x2, j0)
        yj2 = colchunk(y2, j0)
        aj = colchunk(area, j0)
        xi1 = x1[:, i0:i0 + _BLK]
        yi1 = y1[:, i0:i0 + _BLK]
        xi2 = x2[:, i0:i0 + _BLK]
        yi2 = y2[:, i0:i0 + _BLK]
        ai = area[:, i0:i0 + _BLK]
        wx = jnp.maximum(jnp.minimum(xj2, xi2) - jnp.maximum(xj1, xi1), 0.0)
        wy = jnp.maximum(jnp.minimum(yj2, yi2) - jnp.maximum(yj1, yi1), 0.0)
        inter = wx * wy
        union = aj + ai - inter
        return inter / jnp.maximum(union, 1e-9)

    nblk = _KP // _BLK
    for blk in range(nblk):
        i0 = blk * _BLK
        iou = iou_chunk(i0, i0)
        iloc = lax.broadcasted_iota(jnp.int32, (1, _BLK), 1)
        jloc = lax.broadcasted_iota(jnp.int32, (_BLK, 1), 0)
        mbb = jnp.where((iou > _NMS_IOU) & (iloc < jloc), 1.0, 0.0)
        inc = act_ref[0:1, i0:i0 + _BLK]

        def fix_cond(carry):
            _, changed, it = carry
            return changed & (it < _BLK + 2)

        def fix_body(carry):
            a, _, it = carry
            sup = jnp.max(mbb * a, axis=1)
            anew = inc * (1.0 - sup.reshape(1, _BLK))
            return (anew, jnp.any(anew != a), it + 1)

        afin, _, _ = lax.while_loop(
            fix_cond, fix_body, (inc, jnp.bool_(True), jnp.int32(0)))
        act_ref[0:1, i0:i0 + _BLK] = afin
        for jc in range(blk + 1, nblk):
            j0 = jc * _BLK
            iou_p = iou_chunk(j0, i0)
            mp = jnp.where(iou_p > _NMS_IOU, 1.0, 0.0)
            sup = jnp.max(mp * afin, axis=1)
            act_ref[0:1, j0:j0 + _BLK] = (
                act_ref[0:1, j0:j0 + _BLK] * (1.0 - sup.reshape(1, _BLK)))

    keep = (act_ref[:] > 0.5) & (key_ref[0, 0:1, :] > -1e8) & (jlane < _K)
    for cc in range(4):
        out_ref[0, cc:cc + 1, :] = jnp.where(keep, box_ref[0, cc:cc + 1, :], 0.0)


def kernel(images, features, conv_w, conv_b, cls_w, cls_b, reg_w, reg_b):
    f = features[0]
    B, C, gh, gw = f.shape
    img_h, img_w = int(images.shape[-2]), int(images.shape[-1])

    xp = jnp.pad(f, ((0, 0), (0, 0), (1, 1), (1, 1)))
    shifts = [xp[:, :, dy:dy + gh, dx:dx + gw]
              for dy in range(3) for dx in range(3)]
    x9t = jnp.stack(shifts, axis=1).reshape(B, 9 * C, gh * gw)
    w9t = jnp.transpose(conv_w, (0, 2, 3, 1)).reshape(C, 9 * C)
    cb = conv_b.reshape(C, 1)
    whd = jnp.concatenate([
        cls_w.reshape(18, C), reg_w.reshape(36, C),
        jnp.zeros((10, C), jnp.float32)], axis=0)
    hb = jnp.concatenate([cls_b, reg_b, jnp.zeros((10,), jnp.float32)])
    hb = hb.reshape(64, 1)
    aw, ah, acx, acy = _anchor_consts(gh, gw, img_h, img_w)
    aw, ah, acx, acy = map(jnp.asarray, (aw, ah, acx, acy))

    keys, props = pl.pallas_call(
        functools.partial(_head_body, img_h=img_h, img_w=img_w),
        grid=(B,),
        in_specs=[
            pl.BlockSpec((1, 9 * C, gh * gw), lambda b: (b, 0, 0)),
            pl.BlockSpec((C, 9 * C), lambda b: (0, 0)),
            pl.BlockSpec((C, 1), lambda b: (0, 0)),
            pl.BlockSpec((64, C), lambda b: (0, 0)),
            pl.BlockSpec((64, 1), lambda b: (0, 0)),
            pl.BlockSpec((_A, _P), lambda b: (0, 0)),
            pl.BlockSpec((_A, _P), lambda b: (0, 0)),
            pl.BlockSpec((_A, _P), lambda b: (0, 0)),
            pl.BlockSpec((_A, _P), lambda b: (0, 0)),
        ],
        out_specs=[
            pl.BlockSpec((1, _A, _P), lambda b: (b, 0, 0)),
            pl.BlockSpec((1, 4, _A, _P), lambda b: (b, 0, 0, 0)),
        ],
        out_shape=[
            jax.ShapeDtypeStruct((B, _A, _P), jnp.float32),
            jax.ShapeDtypeStruct((B, 4, _A, _P), jnp.float32),
        ],
    )(x9t, w9t, cb, whd, hb, aw, ah, acx, acy)

    kpad = jnp.pad(keys.reshape(B, _N), ((0, 0), (0, _NP - _N)),
                   constant_values=-jnp.inf)

    skey, spos = pl.pallas_call(
        _sort_body,
        out_shape=[
            jax.ShapeDtypeStruct((B, _KP), jnp.float32),
            jax.ShapeDtypeStruct((B, _KP), jnp.int32),
        ],
        scratch_shapes=[
            pltpu.VMEM((B, _NP), jnp.float32),
            pltpu.VMEM((B, _NP), jnp.int32),
        ],
    )(kpad)

    propsf = props.reshape(B, 4, _N)
    boxes = _gather_topk(propsf, spos)

    out_t = pl.pallas_call(
        _nms_body,
        grid=(B,),
        in_specs=[
            pl.BlockSpec((1, 4, _KP), lambda b: (b, 0, 0)),
            pl.BlockSpec((1, 1, _KP), lambda b: (b, 0, 0)),
        ],
        out_specs=pl.BlockSpec((1, 4, _KP), lambda b: (b, 0, 0)),
        out_shape=jax.ShapeDtypeStruct((B, 4, _KP), jnp.float32),
        scratch_shapes=[pltpu.VMEM((1, _KP), jnp.float32)],
    )(boxes, skey.reshape(B, 1, _KP))

    return jnp.transpose(out_t, (0, 2, 1))[:, :_K, :]


# speedup vs baseline: 27.1020x; 1.3292x over previous
import functools

import numpy as np
import jax
import jax.numpy as jnp
from jax import lax
from jax.experimental import pallas as pl
from jax.experimental.pallas import tpu as pltpu
from jax.experimental.pallas import tpu_sc as plsc

_SIZES = (64.0, 256.0, 576.0)
_RATIOS = (0.5, 1.0, 2.0)
_MIN_LEN = 10.0
_NMS_IOU = 0.7
_K = 2000
_KP = 2048
_A = 9
_P = 1024
_N = _A * _P
_NP = 16384
_BLK = 256


def _roll(x, shift, axis):
    return pltpu.roll(x, shift, axis)


def _anchor_consts(gh, gw, img_h, img_w):
    size = np.asarray(_SIZES, np.float32)
    ratio = np.asarray(_RATIOS, np.float32)
    h_ratio = np.sqrt(ratio)
    w_ratio = (np.float32(1.0) / h_ratio).astype(np.float32)
    hs = (h_ratio[:, None] * size[None, :]).reshape(-1)
    ws = (w_ratio[:, None] * size[None, :]).reshape(-1)
    base = np.round(np.stack([-ws, -hs, ws, hs], axis=1) / np.float32(2.0))
    base = base.astype(np.float32)
    stride_h = np.float32(img_h // gh)
    stride_w = np.float32(img_w // gw)
    sx = np.arange(gw, dtype=np.float32) * stride_w
    sy = np.arange(gh, dtype=np.float32) * stride_h
    yy, xx = np.meshgrid(sy, sx, indexing="ij")
    xx = xx.reshape(-1)
    yy = yy.reshape(-1)
    shifts = np.stack([xx, yy, xx, yy], axis=1).astype(np.float32)
    anchors = (shifts[:, None, :] + base[None, :, :]).reshape(-1, 4)
    anchors = anchors.astype(np.float32)
    anc = anchors.reshape(_P, _A, 4).transpose(1, 0, 2)
    x1, y1, x2, y2 = anc[..., 0], anc[..., 1], anc[..., 2], anc[..., 3]
    aw = (x2 - x1) + np.float32(1.0)
    ah = (y2 - y1) + np.float32(1.0)
    acx = x1 + np.float32(0.5) * aw
    acy = y1 + np.float32(0.5) * ah
    return aw, ah, acx, acy


def _head_body(x_ref, w9_ref, cb_ref, whd_ref, hb_ref,
               aw_ref, ah_ref, acx_ref, acy_ref,
               key_ref, prop_ref, *, img_h, img_w):
    y = None
    for t, (dy, dx) in enumerate([(a, b) for a in range(3) for b in range(3)]):
        xs = x_ref[0, :, dy:dy + 32, dx:dx + 32].reshape(256, 1024)
        yt = lax.dot_general(w9_ref[:, t * 256:(t + 1) * 256].astype(jnp.bfloat16),
                             xs.astype(jnp.bfloat16),
                             dimension_numbers=(((1,), (0,)), ((), ())),
                             preferred_element_type=jnp.float32)
        y = yt if y is None else y + yt
    y = jnp.maximum(y + cb_ref[:], 0.0)
    heads = lax.dot_general(whd_ref[:].astype(jnp.bfloat16),
                            y.astype(jnp.bfloat16),
                            dimension_numbers=(((1,), (0,)), ((), ())),
                            preferred_element_type=jnp.float32) + hb_ref[:]
    for a in range(_A):
        x0 = heads[2 * a:2 * a + 1, :]
        x1 = heads[2 * a + 1:2 * a + 2, :]
        r = 18 + 4 * a
        dx = heads[r:r + 1, :]
        dy = heads[r + 1:r + 2, :]
        dw = heads[r + 2:r + 3, :]
        dh = heads[r + 3:r + 4, :]
        aw = aw_ref[pl.ds(a, 1), :]
        ah = ah_ref[pl.ds(a, 1), :]
        acx = acx_ref[pl.ds(a, 1), :]
        acy = acy_ref[pl.ds(a, 1), :]
        pcx = dx * aw + acx
        pcy = dy * ah + acy
        pw = jnp.exp(dw) * aw
        ph = jnp.exp(dh) * ah
        px1 = jnp.maximum(pcx - 0.5 * pw, 0.0)
        py1 = jnp.maximum(pcy - 0.5 * ph, 0.0)
        px2 = jnp.maximum(pcx + 0.5 * pw, 0.0)
        py2 = jnp.maximum(pcy + 0.5 * ph, 0.0)
        cx1 = jnp.clip(px1, 0.0, float(img_w))
        cx2 = jnp.clip(px2, 0.0, float(img_w))
        cy1 = jnp.clip(py1, 0.0, float(img_h))
        cy2 = jnp.clip(py2, 0.0, float(img_h))
        keep = ((cx2 - cx1) >= _MIN_LEN) & ((cy2 - cy1) >= _MIN_LEN)
        m = jnp.maximum(x0, x1)
        e0 = jnp.exp(x0 - m)
        e1 = jnp.exp(x1 - m)
        score = e1 / (e0 + e1)
        key_ref[0, pl.ds(a, 1), :] = jnp.where(keep, score, -1e9)
        prop_ref[0, 0, pl.ds(a, 1), :] = px1
        prop_ref[0, 1, pl.ds(a, 1), :] = py1
        prop_ref[0, 2, pl.ds(a, 1), :] = px2
        prop_ref[0, 3, pl.ds(a, 1), :] = py2


def _sort_body(k_ref, skey_ref, spos_ref, kbuf, cbuf):
    q = lax.broadcasted_iota(jnp.int32, (8, _NP), 1)
    orig = (q % _P) * _A + (q // _P)
    kbuf[:] = k_ref[:]
    cbuf[:] = orig * _NP + q

    def level(lvl, _):
        kk = lax.shift_left(1, lvl)

        def step_cond(j):
            return j >= 1

        def step(j):
            K = kbuf[:]
            C = cbuf[:]
            up_k = _roll(K, j, 1)
            dn_k = _roll(K, _NP - j, 1)
            up_c = _roll(C, j, 1)
            dn_c = _roll(C, _NP - j, 1)
            bit0 = (q & j) == 0
            pk = jnp.where(bit0, dn_k, up_k)
            pc = jnp.where(bit0, dn_c, up_c)
            self_first = (K > pk) | ((K == pk) & (C < pc))
            dirup = (q & kk) == 0
            take_self = self_first == (bit0 == dirup)
            kbuf[:] = jnp.where(take_self, K, pk)
            cbuf[:] = jnp.where(take_self, C, pc)
            return j // 2

        lax.while_loop(step_cond, step, kk // 2)
        return 0

    lax.fori_loop(1, 15, level, 0)
    skey_ref[:] = kbuf[:, :_KP]
    spos_ref[:] = cbuf[:, :_KP] & (_NP - 1)


_GCHUNK = 128


def _sc_gather_body(tab_hbm, posg_hbm, out_hbm, idx_v, rows_v, sem):
    c = lax.axis_index("c")
    s = lax.axis_index("s")
    wid = s * 2 + c
    per_w = (8 * _KP) // 32
    for g in range(per_w // _GCHUNK):
        base = wid * per_w + g * _GCHUNK
        pltpu.sync_copy(posg_hbm.at[pl.ds(base, _GCHUNK)], idx_v)
        pltpu.async_copy(tab_hbm.at[idx_v], rows_v, sem).wait()
        pltpu.sync_copy(rows_v, out_hbm.at[pl.ds(base, _GCHUNK)])


def _gather_topk(propsf, spos):
    B = spos.shape[0]
    tab = jnp.transpose(propsf, (0, 2, 1))
    tab = jnp.pad(tab, ((0, 0), (0, 0), (0, 12))).reshape(B * _N, 16)
    posg = (spos + (jnp.arange(B, dtype=jnp.int32) * _N)[:, None]).reshape(-1)
    mesh = plsc.VectorSubcoreMesh(core_axis_name="c", subcore_axis_name="s")
    run = pl.kernel(
        _sc_gather_body,
        out_type=jax.ShapeDtypeStruct((B * _KP, 16), jnp.float32),
        mesh=mesh,
        compiler_params=pltpu.CompilerParams(use_tc_tiling_on_sc=False),
        scratch_types=[
            pltpu.VMEM((_GCHUNK,), jnp.int32),
            pltpu.VMEM((_GCHUNK, 16), jnp.float32),
            pltpu.SemaphoreType.DMA,
        ],
    )
    rows = run(tab, posg)
    return jnp.transpose(rows.reshape(B, _KP, 16)[:, :, :4], (0, 2, 1))


def _nms_body(box_ref, key_ref, out_ref, act_ref):
    x1 = box_ref[0, 0:1, :]
    y1 = box_ref[0, 1:2, :]
    x2 = box_ref[0, 2:3, :]
    y2 = box_ref[0, 3:4, :]
    area = (x2 - x1) * (y2 - y1)
    jlane = lax.broadcasted_iota(jnp.int32, (1, _KP), 1)
    act_ref[:] = jnp.where(jlane < _K, 1.0, 0.0)

    def colchunk(v, j0):
        return v[:, j0:j0 + _BLK].reshape(_BLK, 1)

    def iou_chunk(j0, i0):
        xj1 = colchunk(x1, j0)
        yj1 = colchunk(y1, j0)
        xj2 = colchunk(x2, j0)
        yj2 = colchunk(y2, j0)
        aj = colchunk(area, j0)
        xi1 = x1[:, i0:i0 + _BLK]
        yi1 = y1[:, i0:i0 + _BLK]
        xi2 = x2[:, i0:i0 + _BLK]
        yi2 = y2[:, i0:i0 + _BLK]
        ai = area[:, i0:i0 + _BLK]
        wx = jnp.maximum(jnp.minimum(xj2, xi2) - jnp.maximum(xj1, xi1), 0.0)
        wy = jnp.maximum(jnp.minimum(yj2, yi2) - jnp.maximum(yj1, yi1), 0.0)
        inter = wx * wy
        union = aj + ai - inter
        return inter / jnp.maximum(union, 1e-9)

    nblk = _KP // _BLK
    for blk in range(nblk):
        i0 = blk * _BLK
        iou = iou_chunk(i0, i0)
        iloc = lax.broadcasted_iota(jnp.int32, (1, _BLK), 1)
        jloc = lax.broadcasted_iota(jnp.int32, (_BLK, 1), 0)
        mbb = jnp.where((iou > _NMS_IOU) & (iloc < jloc), 1.0, 0.0)
        inc = act_ref[0:1, i0:i0 + _BLK]

        def fix_cond(carry):
            _, changed, it = carry
            return changed & (it < _BLK + 2)

        def fix_body(carry):
            a, _, it = carry
            sup = jnp.max(mbb * a, axis=1)
            anew = inc * (1.0 - sup.reshape(1, _BLK))
            return (anew, jnp.any(anew != a), it + 1)

        afin, _, _ = lax.while_loop(
            fix_cond, fix_body, (inc, jnp.bool_(True), jnp.int32(0)))
        act_ref[0:1, i0:i0 + _BLK] = afin
        for jc in range(blk + 1, nblk):
            j0 = jc * _BLK
            iou_p = iou_chunk(j0, i0)
            mp = jnp.where(iou_p > _NMS_IOU, 1.0, 0.0)
            sup = jnp.max(mp * afin, axis=1)
            act_ref[0:1, j0:j0 + _BLK] = (
                act_ref[0:1, j0:j0 + _BLK] * (1.0 - sup.reshape(1, _BLK)))

    keep = (act_ref[:] > 0.5) & (key_ref[0, 0:1, :] > -1e8) & (jlane < _K)
    for cc in range(4):
        out_ref[0, cc:cc + 1, :] = jnp.where(keep, box_ref[0, cc:cc + 1, :], 0.0)


def kernel(images, features, conv_w, conv_b, cls_w, cls_b, reg_w, reg_b):
    f = features[0]
    B, C, gh, gw = f.shape
    img_h, img_w = int(images.shape[-2]), int(images.shape[-1])

    xp = jnp.pad(f, ((0, 0), (0, 0), (1, 1), (1, 1)))
    w9t = jnp.transpose(conv_w, (0, 2, 3, 1)).reshape(C, 9 * C)
    cb = conv_b.reshape(C, 1)
    whd = jnp.concatenate([
        cls_w.reshape(18, C), reg_w.reshape(36, C),
        jnp.zeros((10, C), jnp.float32)], axis=0)
    hb = jnp.concatenate([cls_b, reg_b, jnp.zeros((10,), jnp.float32)])
    hb = hb.reshape(64, 1)
    aw, ah, acx, acy = _anchor_consts(gh, gw, img_h, img_w)
    aw, ah, acx, acy = map(jnp.asarray, (aw, ah, acx, acy))

    keys, props = pl.pallas_call(
        functools.partial(_head_body, img_h=img_h, img_w=img_w),
        grid=(B,),
        in_specs=[
            pl.BlockSpec((1, C, gh + 2, gw + 2), lambda b: (b, 0, 0, 0)),
            pl.BlockSpec((C, 9 * C), lambda b: (0, 0)),
            pl.BlockSpec((C, 1), lambda b: (0, 0)),
            pl.BlockSpec((64, C), lambda b: (0, 0)),
            pl.BlockSpec((64, 1), lambda b: (0, 0)),
            pl.BlockSpec((_A, _P), lambda b: (0, 0)),
            pl.BlockSpec((_A, _P), lambda b: (0, 0)),
            pl.BlockSpec((_A, _P), lambda b: (0, 0)),
            pl.BlockSpec((_A, _P), lambda b: (0, 0)),
        ],
        out_specs=[
            pl.BlockSpec((1, _A, _P), lambda b: (b, 0, 0)),
            pl.BlockSpec((1, 4, _A, _P), lambda b: (b, 0, 0, 0)),
        ],
        out_shape=[
            jax.ShapeDtypeStruct((B, _A, _P), jnp.float32),
            jax.ShapeDtypeStruct((B, 4, _A, _P), jnp.float32),
        ],
    )(xp, w9t, cb, whd, hb, aw, ah, acx, acy)

    kpad = jnp.pad(keys.reshape(B, _N), ((0, 0), (0, _NP - _N)),
                   constant_values=-jnp.inf)

    skey, spos = pl.pallas_call(
        _sort_body,
        out_shape=[
            jax.ShapeDtypeStruct((B, _KP), jnp.float32),
            jax.ShapeDtypeStruct((B, _KP), jnp.int32),
        ],
        scratch_shapes=[
            pltpu.VMEM((B, _NP), jnp.float32),
            pltpu.VMEM((B, _NP), jnp.int32),
        ],
    )(kpad)

    propsf = props.reshape(B, 4, _N)
    boxes = _gather_topk(propsf, spos)

    out_t = pl.pallas_call(
        _nms_body,
        grid=(B,),
        in_specs=[
            pl.BlockSpec((1, 4, _KP), lambda b: (b, 0, 0)),
            pl.BlockSpec((1, 1, _KP), lambda b: (b, 0, 0)),
        ],
        out_specs=pl.BlockSpec((1, 4, _KP), lambda b: (b, 0, 0)),
        out_shape=jax.ShapeDtypeStruct((B, 4, _KP), jnp.float32),
        scratch_shapes=[pltpu.VMEM((1, _KP), jnp.float32)],
    )(boxes, skey.reshape(B, 1, _KP))

    return jnp.transpose(out_t, (0, 2, 1))[:, :_K, :]
